# Initial kernel scaffold; baseline (speedup 1.0000x reference)
import jax
import jax.numpy as jnp
from jax.experimental import pallas as pl


def kernel(action_states, Wq, bq, Wk, bk):
    raise NotImplementedError("write your pallas kernel here")


# speedup vs baseline: 45.9171x; 45.9171x over previous
import functools
import math

import jax
import jax.numpy as jnp
from jax import lax
from jax.experimental import pallas as pl
from jax.experimental.pallas import tpu as pltpu

TOP_K = 8
NEG = -1e30


def _nt_dot(a, b):
    return lax.dot_general(a, b, (((1,), (1,)), ((), ())),
                           preferred_element_type=jnp.float32)


def _dense_body(x_ref, wq_ref, bq_ref, wk_ref, bk_ref, out_ref, k_scratch,
                *, blk_r, scale):
    j = pl.program_id(1)

    @pl.when(j == 0)
    def _():
        k_scratch[...] = _nt_dot(x_ref[0], wk_ref[...]) + bk_ref[...]

    xq = x_ref[0, pl.ds(j * blk_r, blk_r), :]
    q = _nt_dot(xq, wq_ref[...]) + bq_ref[...]
    s = _nt_dot(q, k_scratch[...]) * (1.0 / scale)

    m = jnp.max(s, axis=1, keepdims=True)
    t = m
    for _ in range(TOP_K - 1):
        t = jnp.max(jnp.where(s < t, s, NEG), axis=1, keepdims=True)
    sel = s >= t
    p = jnp.where(sel, jnp.exp(s - m), 0.0)
    denom = jnp.sum(p, axis=1, keepdims=True)
    out_ref[0] = p / denom


def kernel(action_states, Wq, bq, Wk, bk):
    B, N, D = action_states.shape
    DQ = Wq.shape[0]
    scale = math.sqrt(DQ)
    blk_r = 256
    nb = N // blk_r

    body = functools.partial(_dense_body, blk_r=blk_r, scale=scale)
    out = pl.pallas_call(
        body,
        grid=(B, nb),
        in_specs=[
            pl.BlockSpec((1, N, D), lambda b, j: (b, 0, 0)),
            pl.BlockSpec((DQ, D), lambda b, j: (0, 0)),
            pl.BlockSpec((1, DQ), lambda b, j: (0, 0)),
            pl.BlockSpec((DQ, D), lambda b, j: (0, 0)),
            pl.BlockSpec((1, DQ), lambda b, j: (0, 0)),
        ],
        out_specs=pl.BlockSpec((1, blk_r, N), lambda b, j: (b, j, 0)),
        out_shape=jax.ShapeDtypeStruct((B, N, N), jnp.float32),
        scratch_shapes=[pltpu.VMEM((N, DQ), jnp.float32)],
    )(action_states, Wq, bq.reshape(1, DQ), Wk, bk.reshape(1, DQ))
    return out
